# baseline (speedup 1.0000x reference)
import functools

import jax
import jax.numpy as jnp
from jax import lax
from jax.experimental import pallas as pl
from jax.experimental.pallas import tpu as pltpu
from jax.experimental.pallas import tpu_sc as plsc

N_MET = 2534
N_RXN = 4881
N_GENE = 6607
H = 128
N_NZ = 50000
N_PAIR = 20000

NC = 2
NS = 16
NW = NC * NS

METP = 2560
RXNP = 4992
GENEP = 6656
TOTC = RXNP + METP + GENEP
TOTCP = 16384

CH_E = 13
CH_G = 5
CH_C = 30


def _mesh():
    return plsc.VectorSubcoreMesh(core_axis_name="c", subcore_axis_name="s",
                                  num_cores=NC, num_subcores=NS)


def _sc_scatter(table, gidx, sidx, w, zeros2d, *, chunks, nout):
    rp = nout // NS
    weighted = w is not None

    scratch = [
        pltpu.VMEM((chunks, 128), jnp.int32),
        pltpu.VMEM((chunks, 128), jnp.int32),
        pltpu.VMEM((3, 128, H), jnp.float32),
        pltpu.VMEM_SHARED((nout, H), jnp.float32),
        pltpu.SemaphoreType.DMA,
        pltpu.SemaphoreType.DMA,
    ]
    if weighted:
        scratch.insert(2, pltpu.VMEM((chunks, 128), jnp.float32))

    @functools.partial(
        pl.kernel,
        out_type=jax.ShapeDtypeStruct((NC, nout, H), jnp.float32),
        mesh=_mesh(),
        scratch_types=scratch,
        compiler_params=pltpu.CompilerParams(needs_layout_passes=False),
    )
    def k(*refs):
        if weighted:
            (table_hbm, gidx_hbm, sidx_hbm, w_hbm, z_hbm, out_hbm,
             gidx_v, sidx_v, w_v, rows_v, acc, sem, sem2) = refs
        else:
            (table_hbm, gidx_hbm, sidx_hbm, z_hbm, out_hbm,
             gidx_v, sidx_v, rows_v, acc, sem, sem2) = refs
        c = lax.axis_index("c")
        s = lax.axis_index("s")
        wid = s * NC + c
        pltpu.sync_copy(z_hbm.at[pl.ds(0, rp)], acc.at[pl.ds(s * rp, rp)])
        pltpu.sync_copy(gidx_hbm.at[wid], gidx_v)
        pltpu.sync_copy(sidx_hbm.at[wid], sidx_v)
        if weighted:
            pltpu.sync_copy(w_hbm.at[wid], w_v)
        plsc.subcore_barrier()

        pltpu.async_copy(table_hbm.at[gidx_v.at[0]], rows_v.at[0], sem)

        def chunk_body(j, _):
            a = lax.rem(j, 3)
            pltpu.make_async_copy(table_hbm.at[gidx_v.at[j]], rows_v.at[a],
                                  sem).wait()

            @pl.when(j + 1 < chunks)
            def _():
                @pl.when(j >= 2)
                def _():
                    pltpu.make_async_copy(rows_v.at[lax.rem(j - 2, 3)],
                                          acc.at[sidx_v.at[j - 2]],
                                          sem2).wait()
                pltpu.async_copy(table_hbm.at[gidx_v.at[j + 1]],
                                 rows_v.at[lax.rem(j + 1, 3)], sem)

            if weighted:
                def row_body(i, _):
                    sc_ = plsc.load_gather(
                        w_v, [jnp.full((16,), j, jnp.int32),
                              jnp.full((16,), i, jnp.int32)])
                    for cc in range(H // 16):
                        sl = pl.ds(cc * 16, 16)
                        rows_v[a, i, sl] = rows_v[a, i, sl] * sc_
                    return 0

                lax.fori_loop(0, 128, row_body, 0)

            pltpu.async_copy(rows_v.at[a], acc.at[sidx_v.at[j]], sem2,
                             add=True)
            return 0

        lax.fori_loop(0, chunks, chunk_body, 0)
        for t in range(max(chunks - 2, 0), chunks):
            pltpu.make_async_copy(rows_v.at[t % 3], acc.at[sidx_v.at[t]],
                                  sem2).wait()
        plsc.subcore_barrier()
        pltpu.sync_copy(acc.at[pl.ds(s * rp, rp)],
                        out_hbm.at[c, pl.ds(s * rp, rp)])

    if weighted:
        return k(table, gidx, sidx, w, zeros2d)
    return k(table, gidx, sidx, zeros2d)


def _sc_counts(cidx, cw):
    @functools.partial(
        pl.kernel,
        out_type=jax.ShapeDtypeStruct((NW, TOTCP), jnp.float32),
        mesh=_mesh(),
        scratch_types=[
            pltpu.VMEM((CH_C, 128), jnp.int32),
            pltpu.VMEM((CH_C, 128), jnp.float32),
            pltpu.VMEM((TOTCP,), jnp.float32),
        ],
        compiler_params=pltpu.CompilerParams(needs_layout_passes=False),
    )
    def k(cidx_hbm, cw_hbm, out_hbm, idx_v, w_v, hist):
        c = lax.axis_index("c")
        s = lax.axis_index("s")
        wid = s * NC + c
        pltpu.sync_copy(cidx_hbm.at[wid], idx_v)
        pltpu.sync_copy(cw_hbm.at[wid], w_v)

        def zb(i, _):
            hist[pl.ds(i * 16, 16)] = jnp.zeros((16,), jnp.float32)
            return 0

        lax.fori_loop(0, TOTCP // 16, zb, 0)

        def cb(j, _):
            for q in range(8):
                sl = pl.ds(q * 16, 16)
                plsc.addupdate_scatter(hist, [idx_v[j, sl]], w_v[j, sl])
            return 0

        lax.fori_loop(0, CH_C, cb, 0)
        pltpu.sync_copy(hist, out_hbm.at[wid])

    return k(cidx, cw)


def _pre_body(emb_ref, w_ref, out_ref):
    x = emb_ref[...]
    norm = jnp.sqrt(jnp.sum(x * x, axis=1, keepdims=True))
    scale = jnp.minimum(1.0, 1.0 / jnp.maximum(norm, 1e-7))
    out_ref[...] = jnp.dot(x * scale, w_ref[...],
                           preferred_element_type=jnp.float32)


def _tc_pre(emb_p, w0):
    return pl.pallas_call(
        _pre_body,
        out_shape=jax.ShapeDtypeStruct((METP, H), jnp.float32),
    )(emb_p, w0)


def _comb_body(p_ref, c_ref, out_ref):
    inv = 1.0 / jnp.clip(jnp.sum(c_ref[...], axis=0), 1.0, None)
    out_ref[...] = (p_ref[0] + p_ref[1]) * inv[:, None]


def _tc_combine(p, cnt):
    n = p.shape[1]
    return pl.pallas_call(
        _comb_body,
        out_shape=jax.ShapeDtypeStruct((n, H), jnp.float32),
    )(p, cnt)


def _l0_body(q_ref, c_ref, b_ref, w_ref, h_ref, xw_ref):
    inv = 1.0 / jnp.clip(jnp.sum(c_ref[...], axis=0), 1.0, None)
    h = jnp.tanh((q_ref[0] + q_ref[1]) * inv[:, None] + b_ref[...][None, :])
    h_ref[...] = h
    xw_ref[...] = jnp.dot(h, w_ref[...], preferred_element_type=jnp.float32)


def _tc_layer0(q, cnt, b0, w1):
    return pl.pallas_call(
        _l0_body,
        out_shape=[jax.ShapeDtypeStruct((METP, H), jnp.float32),
                   jax.ShapeDtypeStruct((METP, H), jnp.float32)],
    )(q, cnt, b0, w1)


def _l1_body(q_ref, c_ref, b_ref, h_ref, g_ref, be_ref, out_ref):
    inv = 1.0 / jnp.clip(jnp.sum(c_ref[...], axis=0), 1.0, None)
    t = jnp.tanh((q_ref[0] + q_ref[1]) * inv[:, None] + b_ref[...][None, :])
    y = t + h_ref[...]
    mu = jnp.mean(y, axis=1, keepdims=True)
    var = jnp.mean((y - mu) * (y - mu), axis=1, keepdims=True)
    out_ref[...] = ((y - mu) / jnp.sqrt(var + 1e-5) * g_ref[...][None, :]
                    + be_ref[...][None, :])


def _tc_layer1(q, cnt, b1, h, g, be):
    return pl.pallas_call(
        _l1_body,
        out_shape=jax.ShapeDtypeStruct((METP, H), jnp.float32),
    )(q, cnt, b1, h, g, be)


def _padto(x, tot, fill):
    return jnp.pad(x, (0, tot - x.shape[0]), constant_values=fill)


def kernel(hyperedge_index, stoichiometry, gene_x, rxn_gene_rxn_idx,
           rxn_gene_gene_idx, emb_table, W0, b0, W1, b1, ln_gamma, ln_beta):
    del gene_x
    node_idx = hyperedge_index[0].astype(jnp.int32)
    edge_idx = hyperedge_index[1].astype(jnp.int32)
    gr = rxn_gene_rxn_idx.astype(jnp.int32)
    gg = rxn_gene_gene_idx.astype(jnp.int32)

    tot_e = NW * CH_E * 128
    tot_g = NW * CH_G * 128
    sh_e = (NW, CH_E, 128)
    sh_g = (NW, CH_G, 128)
    gA = _padto(node_idx, tot_e, 0).reshape(sh_e)
    sA = _padto(edge_idx, tot_e, 0).reshape(sh_e)
    wA = _padto(stoichiometry, tot_e, 0.0).reshape(sh_e)
    gB = _padto(edge_idx, tot_e, 0).reshape(sh_e)
    sB = _padto(node_idx, tot_e, 0).reshape(sh_e)
    sC = _padto(edge_idx, tot_e, RXNP - 1).reshape(sh_e)
    gD = _padto(gr, tot_g, 0).reshape(sh_g)
    sD = _padto(gg, tot_g, GENEP - 1).reshape(sh_g)

    cidx = jnp.concatenate([edge_idx, node_idx + RXNP, gg + (RXNP + METP)])
    cw = jnp.ones((cidx.shape[0],), jnp.float32)
    tot_c = NW * CH_C * 128
    cidx = _padto(cidx, tot_c, 0).reshape((NW, CH_C, 128))
    cw = _padto(cw, tot_c, 0.0).reshape((NW, CH_C, 128))
    zeros2d = jnp.zeros((512, H), jnp.float32)

    cpart = _sc_counts(cidx, cw)
    cnt_rxn = cpart[:, :RXNP]
    cnt_met = cpart[:, RXNP:RXNP + METP]
    cnt_gene = cpart[:, RXNP + METP:RXNP + METP + GENEP]

    emb_p = jnp.pad(emb_table, ((0, METP - N_MET), (0, 0)))

    xw0 = _tc_pre(emb_p, W0)
    pe0 = _sc_scatter(xw0, gA, sA, wA, zeros2d, chunks=CH_E, nout=RXNP)
    ef0 = _tc_combine(pe0, cnt_rxn)
    pm0 = _sc_scatter(ef0, gB, sB, wA, zeros2d, chunks=CH_E, nout=METP)
    h, xw1 = _tc_layer0(pm0, cnt_met, b0, W1)
    pe1 = _sc_scatter(xw1, gA, sA, wA, zeros2d, chunks=CH_E, nout=RXNP)
    ef1 = _tc_combine(pe1, cnt_rxn)
    pm1 = _sc_scatter(ef1, gB, sB, wA, zeros2d, chunks=CH_E, nout=METP)
    cur = _tc_layer1(pm1, cnt_met, b1, h, ln_gamma, ln_beta)
    pr = _sc_scatter(cur, gA, sC, None, zeros2d, chunks=CH_E, nout=RXNP)
    re = _tc_combine(pr, cnt_rxn)
    pg = _sc_scatter(re, gD, sD, None, zeros2d, chunks=CH_G, nout=GENEP)
    ge = _tc_combine(pg, cnt_gene)
    return ge[:N_GENE]


# speedup vs baseline: 1.0505x; 1.0505x over previous
import functools

import jax
import jax.numpy as jnp
from jax import lax
from jax.experimental import pallas as pl
from jax.experimental.pallas import tpu as pltpu
from jax.experimental.pallas import tpu_sc as plsc

N_MET = 2534
N_RXN = 4881
N_GENE = 6607
H = 128
N_NZ = 50000
N_PAIR = 20000

NC = 2
NS = 16
NW = NC * NS

METP = 2560
RXNP = 4992
GENEP = 6656
TOTC = RXNP + METP + GENEP
TOTCP = 16384

CH_E = 13
CH_G = 5
CH_C = 30


def _mesh():
    return plsc.VectorSubcoreMesh(core_axis_name="c", subcore_axis_name="s",
                                  num_cores=NC, num_subcores=NS)


def _sc_scatter(table, gidx, sidx, w, zeros2d, *, chunks, nout):
    rp = nout // NS
    weighted = w is not None

    scratch = [
        pltpu.VMEM((chunks, 128), jnp.int32),
        pltpu.VMEM((chunks, 128), jnp.int32),
        pltpu.VMEM((2, 128, H), jnp.float32),
        pltpu.VMEM_SHARED((nout, H), jnp.float32),
        pltpu.SemaphoreType.DMA,
    ]
    if weighted:
        scratch.insert(2, pltpu.VMEM((chunks, 128), jnp.float32))

    @functools.partial(
        pl.kernel,
        out_type=jax.ShapeDtypeStruct((NC, nout, H), jnp.float32),
        mesh=_mesh(),
        scratch_types=scratch,
        compiler_params=pltpu.CompilerParams(needs_layout_passes=False),
    )
    def k(*refs):
        if weighted:
            (table_hbm, gidx_hbm, sidx_hbm, w_hbm, z_hbm, out_hbm,
             gidx_v, sidx_v, w_v, rows_v, acc, sem) = refs
        else:
            (table_hbm, gidx_hbm, sidx_hbm, z_hbm, out_hbm,
             gidx_v, sidx_v, rows_v, acc, sem) = refs
        c = lax.axis_index("c")
        s = lax.axis_index("s")
        wid = s * NC + c
        pltpu.sync_copy(gidx_hbm.at[wid], gidx_v)
        pltpu.async_copy(table_hbm.at[gidx_v.at[0]], rows_v.at[0], sem)
        pltpu.sync_copy(sidx_hbm.at[wid], sidx_v)
        if weighted:
            pltpu.sync_copy(w_hbm.at[wid], w_v)
        pltpu.sync_copy(z_hbm.at[pl.ds(0, rp)], acc.at[pl.ds(s * rp, rp)])
        plsc.subcore_barrier()

        def chunk_body(j, _):
            p = lax.rem(j, 2)
            pltpu.make_async_copy(table_hbm.at[gidx_v.at[j]], rows_v.at[p],
                                  sem).wait()

            @pl.when(j + 1 < chunks)
            def _():
                pltpu.async_copy(table_hbm.at[gidx_v.at[j + 1]],
                                 rows_v.at[1 - p], sem)

            if weighted:
                def row_body(i, _):
                    sc_ = plsc.load_gather(
                        w_v, [jnp.full((16,), j, jnp.int32),
                              jnp.full((16,), i, jnp.int32)])
                    for cc in range(H // 16):
                        sl = pl.ds(cc * 16, 16)
                        rows_v[p, i, sl] = rows_v[p, i, sl] * sc_
                    return 0

                lax.fori_loop(0, 128, row_body, 0)

            pltpu.sync_copy(rows_v.at[p], acc.at[sidx_v.at[j]], add=True)
            return 0

        lax.fori_loop(0, chunks, chunk_body, 0)
        plsc.subcore_barrier()
        pltpu.sync_copy(acc.at[pl.ds(s * rp, rp)],
                        out_hbm.at[c, pl.ds(s * rp, rp)])

    if weighted:
        return k(table, gidx, sidx, w, zeros2d)
    return k(table, gidx, sidx, zeros2d)


def _sc_counts(cidx, cw):
    @functools.partial(
        pl.kernel,
        out_type=jax.ShapeDtypeStruct((NW, TOTCP), jnp.float32),
        mesh=_mesh(),
        scratch_types=[
            pltpu.VMEM((CH_C, 128), jnp.int32),
            pltpu.VMEM((CH_C, 128), jnp.float32),
            pltpu.VMEM((TOTCP,), jnp.float32),
        ],
        compiler_params=pltpu.CompilerParams(needs_layout_passes=False),
    )
    def k(cidx_hbm, cw_hbm, out_hbm, idx_v, w_v, hist):
        c = lax.axis_index("c")
        s = lax.axis_index("s")
        wid = s * NC + c
        pltpu.sync_copy(cidx_hbm.at[wid], idx_v)
        pltpu.sync_copy(cw_hbm.at[wid], w_v)

        def zb(i, _):
            hist[pl.ds(i * 16, 16)] = jnp.zeros((16,), jnp.float32)
            return 0

        lax.fori_loop(0, TOTCP // 16, zb, 0)

        def cb(j, _):
            for q in range(8):
                sl = pl.ds(q * 16, 16)
                plsc.addupdate_scatter(hist, [idx_v[j, sl]], w_v[j, sl])
            return 0

        lax.fori_loop(0, CH_C, cb, 0)
        pltpu.sync_copy(hist, out_hbm.at[wid])

    return k(cidx, cw)


def _pre_body(emb_ref, w_ref, out_ref):
    x = emb_ref[...]
    norm = jnp.sqrt(jnp.sum(x * x, axis=1, keepdims=True))
    scale = jnp.minimum(1.0, 1.0 / jnp.maximum(norm, 1e-7))
    out_ref[...] = jnp.dot(x * scale, w_ref[...],
                           preferred_element_type=jnp.float32)


def _tc_pre(emb_p, w0):
    return pl.pallas_call(
        _pre_body,
        out_shape=jax.ShapeDtypeStruct((METP, H), jnp.float32),
    )(emb_p, w0)


def _comb_body(p_ref, c_ref, out_ref):
    inv = 1.0 / jnp.clip(jnp.sum(c_ref[...], axis=0), 1.0, None)
    out_ref[...] = (p_ref[0] + p_ref[1]) * inv[:, None]


def _tc_combine(p, cnt):
    n = p.shape[1]
    return pl.pallas_call(
        _comb_body,
        out_shape=jax.ShapeDtypeStruct((n, H), jnp.float32),
    )(p, cnt)


def _l0_body(q_ref, c_ref, b_ref, w_ref, h_ref, xw_ref):
    inv = 1.0 / jnp.clip(jnp.sum(c_ref[...], axis=0), 1.0, None)
    h = jnp.tanh((q_ref[0] + q_ref[1]) * inv[:, None] + b_ref[...][None, :])
    h_ref[...] = h
    xw_ref[...] = jnp.dot(h, w_ref[...], preferred_element_type=jnp.float32)


def _tc_layer0(q, cnt, b0, w1):
    return pl.pallas_call(
        _l0_body,
        out_shape=[jax.ShapeDtypeStruct((METP, H), jnp.float32),
                   jax.ShapeDtypeStruct((METP, H), jnp.float32)],
    )(q, cnt, b0, w1)


def _l1_body(q_ref, c_ref, b_ref, h_ref, g_ref, be_ref, out_ref):
    inv = 1.0 / jnp.clip(jnp.sum(c_ref[...], axis=0), 1.0, None)
    t = jnp.tanh((q_ref[0] + q_ref[1]) * inv[:, None] + b_ref[...][None, :])
    y = t + h_ref[...]
    mu = jnp.mean(y, axis=1, keepdims=True)
    var = jnp.mean((y - mu) * (y - mu), axis=1, keepdims=True)
    out_ref[...] = ((y - mu) / jnp.sqrt(var + 1e-5) * g_ref[...][None, :]
                    + be_ref[...][None, :])


def _tc_layer1(q, cnt, b1, h, g, be):
    return pl.pallas_call(
        _l1_body,
        out_shape=jax.ShapeDtypeStruct((METP, H), jnp.float32),
    )(q, cnt, b1, h, g, be)


def _padto(x, tot, fill):
    return jnp.pad(x, (0, tot - x.shape[0]), constant_values=fill)


def kernel(hyperedge_index, stoichiometry, gene_x, rxn_gene_rxn_idx,
           rxn_gene_gene_idx, emb_table, W0, b0, W1, b1, ln_gamma, ln_beta):
    del gene_x
    node_idx = hyperedge_index[0].astype(jnp.int32)
    edge_idx = hyperedge_index[1].astype(jnp.int32)
    gr = rxn_gene_rxn_idx.astype(jnp.int32)
    gg = rxn_gene_gene_idx.astype(jnp.int32)

    tot_e = NW * CH_E * 128
    tot_g = NW * CH_G * 128
    sh_e = (NW, CH_E, 128)
    sh_g = (NW, CH_G, 128)
    gA = _padto(node_idx, tot_e, 0).reshape(sh_e)
    sA = _padto(edge_idx, tot_e, 0).reshape(sh_e)
    wA = _padto(stoichiometry, tot_e, 0.0).reshape(sh_e)
    gB = _padto(edge_idx, tot_e, 0).reshape(sh_e)
    sB = _padto(node_idx, tot_e, 0).reshape(sh_e)
    sC = _padto(edge_idx, tot_e, RXNP - 1).reshape(sh_e)
    gD = _padto(gr, tot_g, 0).reshape(sh_g)
    sD = _padto(gg, tot_g, GENEP - 1).reshape(sh_g)

    cidx = jnp.concatenate([edge_idx, node_idx + RXNP, gg + (RXNP + METP)])
    cw = jnp.ones((cidx.shape[0],), jnp.float32)
    tot_c = NW * CH_C * 128
    cidx = _padto(cidx, tot_c, 0).reshape((NW, CH_C, 128))
    cw = _padto(cw, tot_c, 0.0).reshape((NW, CH_C, 128))
    zeros2d = jnp.zeros((512, H), jnp.float32)

    cpart = _sc_counts(cidx, cw)
    cnt_rxn = cpart[:, :RXNP]
    cnt_met = cpart[:, RXNP:RXNP + METP]
    cnt_gene = cpart[:, RXNP + METP:RXNP + METP + GENEP]

    emb_p = jnp.pad(emb_table, ((0, METP - N_MET), (0, 0)))

    xw0 = _tc_pre(emb_p, W0)
    pe0 = _sc_scatter(xw0, gA, sA, wA, zeros2d, chunks=CH_E, nout=RXNP)
    ef0 = _tc_combine(pe0, cnt_rxn)
    pm0 = _sc_scatter(ef0, gB, sB, wA, zeros2d, chunks=CH_E, nout=METP)
    h, xw1 = _tc_layer0(pm0, cnt_met, b0, W1)
    pe1 = _sc_scatter(xw1, gA, sA, wA, zeros2d, chunks=CH_E, nout=RXNP)
    ef1 = _tc_combine(pe1, cnt_rxn)
    pm1 = _sc_scatter(ef1, gB, sB, wA, zeros2d, chunks=CH_E, nout=METP)
    cur = _tc_layer1(pm1, cnt_met, b1, h, ln_gamma, ln_beta)
    pr = _sc_scatter(cur, gA, sC, None, zeros2d, chunks=CH_E, nout=RXNP)
    re = _tc_combine(pr, cnt_rxn)
    pg = _sc_scatter(re, gD, sD, None, zeros2d, chunks=CH_G, nout=GENEP)
    ge = _tc_combine(pg, cnt_gene)
    return ge[:N_GENE]
